# baseline (speedup 1.0000x reference)
import functools

import jax
import jax.numpy as jnp
from jax import lax
from jax.experimental import pallas as pl
from jax.experimental.pallas import tpu as pltpu
from jax.experimental.pallas import tpu_sc as plsc

NC = 2
NS = 16
NW = NC * NS
CHW = 128


def _sc_mesh():
    return plsc.VectorSubcoreMesh(core_axis_name="c", subcore_axis_name="s")


def _make_degree_kernel(n_pad, ep):
    def body(dst_hbm, ones_hbm, zeros_hbm, out_hbm, dstv, onesv, acc):
        c = lax.axis_index("c")
        s = lax.axis_index("s")
        t = c * NS + s
        r = n_pad // NS
        pltpu.sync_copy(zeros_hbm.at[pl.ds(s * r, r)], acc.at[pl.ds(s * r, r)])
        pltpu.sync_copy(dst_hbm.at[t], dstv)
        pltpu.sync_copy(ones_hbm, onesv)
        plsc.subcore_barrier()

        def chunk(j, carry):
            pltpu.sync_copy(onesv, acc.at[dstv.at[j]], add=True)
            return carry

        lax.fori_loop(0, ep, chunk, 0)
        plsc.subcore_barrier()
        pltpu.sync_copy(acc.at[pl.ds(s * r, r)], out_hbm.at[c, pl.ds(s * r, r)])

    return pl.kernel(
        body,
        out_type=jax.ShapeDtypeStruct((NC, n_pad, 8), jnp.float32),
        mesh=_sc_mesh(),
        compiler_params=pltpu.CompilerParams(use_tc_tiling_on_sc=False),
        scratch_types=[
            pltpu.VMEM((ep, CHW), jnp.int32),
            pltpu.VMEM((CHW, 8), jnp.float32),
            pltpu.VMEM_SHARED((n_pad, 8), jnp.float32),
        ],
    )


NB = 4


def _make_segsum_kernel(n_pad, ep, f):
    assert ep % (2 * NB) == 0
    npairs = ep // (2 * NB)

    def body(g_hbm, src_hbm, dst_hbm, zeros_hbm, out_hbm, srcv, dstv,
             msga, msgb, acc, gsem, ssem):
        c = lax.axis_index("c")
        s = lax.axis_index("s")
        t = c * NS + s
        r = n_pad // NS
        pltpu.sync_copy(zeros_hbm.at[pl.ds(s * r, r)], acc.at[pl.ds(s * r, r)])
        pltpu.sync_copy(src_hbm.at[t], srcv)
        pltpu.sync_copy(dst_hbm.at[t], dstv)
        plsc.subcore_barrier()

        def fire_gathers(base, msg):
            for b in range(NB):
                pltpu.async_copy(g_hbm.at[srcv.at[base + b]], msg.at[b], gsem)

        def drain_gathers(base, msg):
            for b in range(NB):
                pltpu.make_async_copy(
                    g_hbm.at[srcv.at[base + b]], msg.at[b], gsem).wait()

        def fire_scatters(base, msg):
            for b in range(NB):
                pltpu.async_copy(
                    msg.at[b], acc.at[dstv.at[base + b]], ssem, add=True)

        def drain_scatters(base, msg):
            for b in range(NB):
                pltpu.make_async_copy(
                    msg.at[b], acc.at[dstv.at[base + b]], ssem).wait()

        def chunk(j, carry):
            pltpu.sync_copy(g_hbm.at[srcv.at[j]], msga.at[0])
            pltpu.sync_copy(msga.at[0], acc.at[dstv.at[j]], add=True)
            return carry

        lax.fori_loop(0, ep, chunk, 0)

        plsc.subcore_barrier()
        pltpu.sync_copy(acc.at[pl.ds(s * r, r)], out_hbm.at[c, pl.ds(s * r, r)])

    return pl.kernel(
        body,
        out_type=jax.ShapeDtypeStruct((NC, n_pad, f), jnp.float32),
        mesh=_sc_mesh(),
        compiler_params=pltpu.CompilerParams(use_tc_tiling_on_sc=False),
        scratch_types=[
            pltpu.VMEM((ep, CHW), jnp.int32),
            pltpu.VMEM((ep, CHW), jnp.int32),
            pltpu.VMEM((NB, CHW, f), jnp.float32),
            pltpu.VMEM((NB, CHW, f), jnp.float32),
            pltpu.VMEM_SHARED((n_pad, f), jnp.float32),
            pltpu.SemaphoreType.DMA,
            pltpu.SemaphoreType.DMA,
        ],
    )


def _tc1_body(x_ref, w_ref, dp_ref, g_ref, dv_ref):
    deg = dp_ref[0, :, 0:1] + dp_ref[1, :, 0:1] + 1.0
    dinv = lax.rsqrt(jnp.maximum(deg, 1.0))
    h = jnp.dot(x_ref[...], w_ref[...], preferred_element_type=jnp.float32)
    g_ref[...] = h * dinv
    dv_ref[...] = jnp.broadcast_to(dinv, dv_ref.shape)


def _tc2_body(sp_ref, g1_ref, dv_ref, b1_ref, w2_ref, g2_ref):
    dinv = dv_ref[:, 0:1]
    h1 = jnp.maximum(
        dinv * (sp_ref[0] + sp_ref[1] + g1_ref[...]) + b1_ref[...], 0.0)
    g2_ref[...] = jnp.dot(h1, w2_ref[...], preferred_element_type=jnp.float32) * dinv


def _tc3_body(sp_ref, g2_ref, dv_ref, b2_ref, o_ref):
    dinv = dv_ref[:, 0:1]
    t = dinv * (sp_ref[0] + sp_ref[1] + g2_ref[...]) + b2_ref[...]
    m = jnp.max(t, axis=1, keepdims=True)
    lse = m + jnp.log(jnp.sum(jnp.exp(t - m), axis=1, keepdims=True))
    o_ref[...] = t - lse


def _rows_spec(bn, cols):
    return pl.BlockSpec((bn, cols), lambda i: (i, 0))


def _pair_spec(bn, cols):
    return pl.BlockSpec((2, bn, cols), lambda i: (0, i, 0))


def _full_spec(shape):
    return pl.BlockSpec(shape, lambda i: tuple(0 for _ in shape))


@jax.jit
def kernel(x, edge_index, W1, b1, W2, b2):
    n, f_in = x.shape
    h = W1.shape[1]
    c_out = W2.shape[1]
    e = edge_index.shape[1]

    n_pad = ((n + 1) + NS * 8 - 1) // (NS * 8) * (NS * 8)
    dump = n
    ep = (e + NW * CHW - 1) // (NW * CHW)
    ep = (ep + 2 * NB - 1) // (2 * NB) * (2 * NB)
    e_pad = ep * NW * CHW

    src = edge_index[0]
    dst = edge_index[1]
    src_r = jnp.concatenate(
        [src, jnp.zeros((e_pad - e,), jnp.int32)]).reshape(NW, ep, CHW)
    dst_r = jnp.concatenate(
        [dst, jnp.full((e_pad - e,), dump, jnp.int32)]).reshape(NW, ep, CHW)

    zeros8 = jnp.zeros((n_pad, 8), jnp.float32)
    ones8 = jnp.ones((CHW, 8), jnp.float32)

    degp = _make_degree_kernel(n_pad, ep)(dst_r, ones8, zeros8)

    bn = 2000
    grid = (n // bn,)
    g1, dv = pl.pallas_call(
        _tc1_body,
        grid=grid,
        in_specs=[
            _rows_spec(bn, f_in),
            _full_spec((f_in, h)),
            _pair_spec(bn, 8),
        ],
        out_specs=[_rows_spec(bn, h), _rows_spec(bn, 8)],
        out_shape=[
            jax.ShapeDtypeStruct((n, h), jnp.float32),
            jax.ShapeDtypeStruct((n, 8), jnp.float32),
        ],
    )(x, W1, degp)

    zeros_h = jnp.zeros((n_pad, h), jnp.float32)
    sp1 = _make_segsum_kernel(n_pad, ep, h)(g1, src_r, dst_r, zeros_h)

    g2 = pl.pallas_call(
        _tc2_body,
        grid=grid,
        in_specs=[
            _pair_spec(bn, h),
            _rows_spec(bn, h),
            _rows_spec(bn, 8),
            _full_spec((1, h)),
            _full_spec((h, c_out)),
        ],
        out_specs=_rows_spec(bn, c_out),
        out_shape=jax.ShapeDtypeStruct((n, c_out), jnp.float32),
    )(sp1, g1, dv, b1.reshape(1, h), W2)

    zeros_c = jnp.zeros((n_pad, c_out), jnp.float32)
    sp2 = _make_segsum_kernel(n_pad, ep, c_out)(g2, src_r, dst_r, zeros_c)

    out = pl.pallas_call(
        _tc3_body,
        grid=grid,
        in_specs=[
            _pair_spec(bn, c_out),
            _rows_spec(bn, c_out),
            _rows_spec(bn, 8),
            _full_spec((1, c_out)),
        ],
        out_specs=_rows_spec(bn, c_out),
        out_shape=jax.ShapeDtypeStruct((n, c_out), jnp.float32),
    )(sp2, g2, dv, b2.reshape(1, c_out))

    return out


# speedup vs baseline: 1.2166x; 1.2166x over previous
import jax
import jax.numpy as jnp
from jax import lax
from jax.experimental import pallas as pl
from jax.experimental.pallas import tpu as pltpu
from jax.experimental.pallas import tpu_sc as plsc

NC = 2
NS = 16
NW = NC * NS
CHW = 128


def _sc_mesh():
    return plsc.VectorSubcoreMesh(core_axis_name="c", subcore_axis_name="s")


def _make_degree_kernel(n_pad, ep):
    def body(edges_hbm, ones_hbm, zeros_hbm, out_hbm, dstv, onesv, acc):
        c = lax.axis_index("c")
        s = lax.axis_index("s")
        t = c * NS + s
        r = n_pad // NS
        pltpu.sync_copy(zeros_hbm.at[pl.ds(s * r, r)], acc.at[pl.ds(s * r, r)])
        pltpu.sync_copy(edges_hbm.at[1, t], dstv)
        pltpu.sync_copy(ones_hbm, onesv)
        plsc.subcore_barrier()

        def chunk(j, carry):
            pltpu.sync_copy(onesv, acc.at[dstv.at[j]], add=True)
            return carry

        lax.fori_loop(0, ep, chunk, 0)
        plsc.subcore_barrier()
        pltpu.sync_copy(acc.at[pl.ds(s * r, r)], out_hbm.at[c, pl.ds(s * r, r)])

    return pl.kernel(
        body,
        out_type=jax.ShapeDtypeStruct((NC, n_pad, 8), jnp.float32),
        mesh=_sc_mesh(),
        compiler_params=pltpu.CompilerParams(use_tc_tiling_on_sc=False),
        scratch_types=[
            pltpu.VMEM((ep, CHW), jnp.int32),
            pltpu.VMEM((CHW, 8), jnp.float32),
            pltpu.VMEM_SHARED((n_pad, 8), jnp.float32),
        ],
    )


def _make_segsum_kernel(n_pad, ep, f):
    def body(g_hbm, edges_hbm, zeros_hbm, out_hbm, srcv, dstv, msg, acc):
        c = lax.axis_index("c")
        s = lax.axis_index("s")
        t = c * NS + s
        r = n_pad // NS
        pltpu.sync_copy(zeros_hbm.at[pl.ds(s * r, r)], acc.at[pl.ds(s * r, r)])
        pltpu.sync_copy(edges_hbm.at[0, t], srcv)
        pltpu.sync_copy(edges_hbm.at[1, t], dstv)
        plsc.subcore_barrier()

        def chunk(j, carry):
            pltpu.sync_copy(g_hbm.at[srcv.at[j]], msg)
            pltpu.sync_copy(msg, acc.at[dstv.at[j]], add=True)
            return carry

        lax.fori_loop(0, ep, chunk, 0)
        plsc.subcore_barrier()
        pltpu.sync_copy(acc.at[pl.ds(s * r, r)], out_hbm.at[c, pl.ds(s * r, r)])

    return pl.kernel(
        body,
        out_type=jax.ShapeDtypeStruct((NC, n_pad, f), jnp.float32),
        mesh=_sc_mesh(),
        compiler_params=pltpu.CompilerParams(use_tc_tiling_on_sc=False),
        scratch_types=[
            pltpu.VMEM((ep, CHW), jnp.int32),
            pltpu.VMEM((ep, CHW), jnp.int32),
            pltpu.VMEM((CHW, f), jnp.float32),
            pltpu.VMEM_SHARED((n_pad, f), jnp.float32),
        ],
    )


def _dinv(d0_ref, d1_ref):
    deg = d0_ref[:, 0:1] + d1_ref[:, 0:1] + 1.0
    return lax.rsqrt(jnp.maximum(deg, 1.0))


def _mm_body(x_ref, w_ref, h_ref):
    h_ref[...] = jnp.dot(x_ref[...], w_ref[...],
                         preferred_element_type=jnp.float32)


def _tc1_body(h_ref, d0_ref, d1_ref, g_ref):
    g_ref[...] = h_ref[...] * _dinv(d0_ref, d1_ref)


def _tc2_body(p0_ref, p1_ref, g1_ref, d0_ref, d1_ref, b1_ref, w2_ref, g2_ref):
    dinv = _dinv(d0_ref, d1_ref)
    h1 = jnp.maximum(
        dinv * (p0_ref[...] + p1_ref[...] + g1_ref[...]) + b1_ref[...], 0.0)
    g2_ref[...] = jnp.dot(h1, w2_ref[...],
                          preferred_element_type=jnp.float32) * dinv


def _tc3_body(q0_ref, q1_ref, g2_ref, d0_ref, d1_ref, b2_ref, o_ref):
    t = (_dinv(d0_ref, d1_ref) * (q0_ref[...] + q1_ref[...] + g2_ref[...])
         + b2_ref[...])
    m = jnp.max(t, axis=1, keepdims=True)
    lse = m + jnp.log(jnp.sum(jnp.exp(t - m), axis=1, keepdims=True))
    o_ref[...] = t - lse


def _rows_spec(bn, cols):
    return pl.BlockSpec((bn, cols), lambda i: (i, 0))


def _full_spec(shape):
    return pl.BlockSpec(shape, lambda i: tuple(0 for _ in shape))


@jax.jit
def kernel(x, edge_index, W1, b1, W2, b2):
    n, f_in = x.shape
    h = W1.shape[1]
    c_out = W2.shape[1]
    e = edge_index.shape[1]

    n_pad = ((n + 1) + NS * 8 - 1) // (NS * 8) * (NS * 8)
    dump = n
    ep = (e + NW * CHW - 1) // (NW * CHW)
    e_pad = ep * NW * CHW

    pad_col = jnp.concatenate([
        jnp.zeros((1, e_pad - e), jnp.int32),
        jnp.full((1, e_pad - e), dump, jnp.int32),
    ])
    edges = jnp.concatenate([edge_index, pad_col], axis=1).reshape(
        2, NW, ep, CHW)

    zeros8 = jnp.zeros((n_pad, 8), jnp.float32)
    ones8 = jnp.ones((CHW, 8), jnp.float32)

    bn = 1000
    grid = (n // bn,)

    h1m = pl.pallas_call(
        _mm_body,
        grid=grid,
        in_specs=[_rows_spec(bn, f_in), _full_spec((f_in, h))],
        out_specs=_rows_spec(bn, h),
        out_shape=jax.ShapeDtypeStruct((n, h), jnp.float32),
    )(x, W1)

    degp = _make_degree_kernel(n_pad, ep)(edges, ones8, zeros8)
    d0 = degp[0, :n]
    d1 = degp[1, :n]

    g1 = pl.pallas_call(
        _tc1_body,
        grid=grid,
        in_specs=[
            _rows_spec(bn, h),
            _rows_spec(bn, 8),
            _rows_spec(bn, 8),
        ],
        out_specs=_rows_spec(bn, h),
        out_shape=jax.ShapeDtypeStruct((n, h), jnp.float32),
    )(h1m, d0, d1)

    zeros_h = jnp.zeros((n_pad, h), jnp.float32)
    sp1 = _make_segsum_kernel(n_pad, ep, h)(g1, edges, zeros_h)

    g2 = pl.pallas_call(
        _tc2_body,
        grid=grid,
        in_specs=[
            _rows_spec(bn, h),
            _rows_spec(bn, h),
            _rows_spec(bn, h),
            _rows_spec(bn, 8),
            _rows_spec(bn, 8),
            _full_spec((1, h)),
            _full_spec((h, c_out)),
        ],
        out_specs=_rows_spec(bn, c_out),
        out_shape=jax.ShapeDtypeStruct((n, c_out), jnp.float32),
    )(sp1[0, :n], sp1[1, :n], g1, d0, d1, b1.reshape(1, h), W2)

    zeros_c = jnp.zeros((n_pad, c_out), jnp.float32)
    sp2 = _make_segsum_kernel(n_pad, ep, c_out)(g2, edges, zeros_c)

    out = pl.pallas_call(
        _tc3_body,
        grid=grid,
        in_specs=[
            _rows_spec(bn, c_out),
            _rows_spec(bn, c_out),
            _rows_spec(bn, c_out),
            _rows_spec(bn, 8),
            _rows_spec(bn, 8),
            _full_spec((1, c_out)),
        ],
        out_specs=_rows_spec(bn, c_out),
        out_shape=jax.ShapeDtypeStruct((n, c_out), jnp.float32),
    )(sp2[0, :n], sp2[1, :n], g2, d0, d1, b2.reshape(1, c_out))

    return out


# speedup vs baseline: 1.2790x; 1.0513x over previous
import jax
import jax.numpy as jnp
from jax import lax
from jax.experimental import pallas as pl
from jax.experimental.pallas import tpu as pltpu
from jax.experimental.pallas import tpu_sc as plsc

NC = 2
NS = 16
NW = NC * NS
CHW = 128

K0 = 91
K1 = 66
KMAX = max(K0, K1)


def _sc_mesh():
    return plsc.VectorSubcoreMesh(core_axis_name="c", subcore_axis_name="s")


def _make_degree_kernel(n_pad):
    def body(edges_hbm, ones_hbm, zeros_hbm, out_hbm, dstv, onesv, acc):
        c = lax.axis_index("c")
        s = lax.axis_index("s")
        k = jnp.where(c == 0, K0, K1)
        start = jnp.where(c == 0, s * K0, NS * K0 + s * K1)
        r = n_pad // NS
        pltpu.sync_copy(zeros_hbm.at[pl.ds(s * r, r)], acc.at[pl.ds(s * r, r)])
        pltpu.sync_copy(edges_hbm.at[1, pl.ds(start, KMAX)], dstv)
        pltpu.sync_copy(ones_hbm, onesv)
        plsc.subcore_barrier()

        def chunk(j, carry):
            @pl.when(j < k)
            def _():
                pltpu.sync_copy(onesv, acc.at[dstv.at[j]], add=True)
            return carry

        lax.fori_loop(0, KMAX, chunk, 0)
        plsc.subcore_barrier()
        pltpu.sync_copy(acc.at[pl.ds(s * r, r)], out_hbm.at[c, pl.ds(s * r, r)])

    return pl.kernel(
        body,
        out_type=jax.ShapeDtypeStruct((NC, n_pad, 8), jnp.float32),
        mesh=_sc_mesh(),
        compiler_params=pltpu.CompilerParams(use_tc_tiling_on_sc=False),
        scratch_types=[
            pltpu.VMEM((KMAX, CHW), jnp.int32),
            pltpu.VMEM((CHW, 8), jnp.float32),
            pltpu.VMEM_SHARED((n_pad, 8), jnp.float32),
        ],
    )


def _make_segsum_kernel(n_pad, f):
    def body(g_hbm, edges_hbm, zeros_hbm, out_hbm, srcv, dstv, msg, acc):
        c = lax.axis_index("c")
        s = lax.axis_index("s")
        k = jnp.where(c == 0, K0, K1)
        start = jnp.where(c == 0, s * K0, NS * K0 + s * K1)
        r = n_pad // NS
        pltpu.sync_copy(zeros_hbm.at[pl.ds(s * r, r)], acc.at[pl.ds(s * r, r)])
        pltpu.sync_copy(edges_hbm.at[0, pl.ds(start, KMAX)], srcv)
        pltpu.sync_copy(edges_hbm.at[1, pl.ds(start, KMAX)], dstv)
        plsc.subcore_barrier()

        def chunk(j, carry):
            @pl.when(j < k)
            def _():
                pltpu.sync_copy(g_hbm.at[srcv.at[j]], msg)
                pltpu.sync_copy(msg, acc.at[dstv.at[j]], add=True)
            return carry

        lax.fori_loop(0, KMAX, chunk, 0)
        plsc.subcore_barrier()
        pltpu.sync_copy(acc.at[pl.ds(s * r, r)], out_hbm.at[c, pl.ds(s * r, r)])

    return pl.kernel(
        body,
        out_type=jax.ShapeDtypeStruct((NC, n_pad, f), jnp.float32),
        mesh=_sc_mesh(),
        compiler_params=pltpu.CompilerParams(use_tc_tiling_on_sc=False),
        scratch_types=[
            pltpu.VMEM((KMAX, CHW), jnp.int32),
            pltpu.VMEM((KMAX, CHW), jnp.int32),
            pltpu.VMEM((CHW, f), jnp.float32),
            pltpu.VMEM_SHARED((n_pad, f), jnp.float32),
        ],
    )


def _dinv(d0_ref, d1_ref):
    deg = d0_ref[:, 0:1] + d1_ref[:, 0:1] + 1.0
    return lax.rsqrt(jnp.maximum(deg, 1.0))


def _mm_body(x_ref, w_ref, h_ref):
    h_ref[...] = jnp.dot(x_ref[...], w_ref[...],
                         preferred_element_type=jnp.float32)


def _tc1_body(h_ref, d0_ref, d1_ref, g_ref):
    g_ref[...] = h_ref[...] * _dinv(d0_ref, d1_ref)


def _tc2_body(p0_ref, p1_ref, g1_ref, d0_ref, d1_ref, b1_ref, w2_ref, g2_ref):
    dinv = _dinv(d0_ref, d1_ref)
    h1 = jnp.maximum(
        dinv * (p0_ref[...] + p1_ref[...] + g1_ref[...]) + b1_ref[...], 0.0)
    g2_ref[...] = jnp.dot(h1, w2_ref[...],
                          preferred_element_type=jnp.float32) * dinv


def _tc3_body(q0_ref, q1_ref, g2_ref, d0_ref, d1_ref, b2_ref, o_ref):
    t = (_dinv(d0_ref, d1_ref) * (q0_ref[...] + q1_ref[...] + g2_ref[...])
         + b2_ref[...])
    m = jnp.max(t, axis=1, keepdims=True)
    lse = m + jnp.log(jnp.sum(jnp.exp(t - m), axis=1, keepdims=True))
    o_ref[...] = t - lse


def _rows_spec(bn, cols):
    return pl.BlockSpec((bn, cols), lambda i: (i, 0))


def _full_spec(shape):
    return pl.BlockSpec(shape, lambda i: tuple(0 for _ in shape))


@jax.jit
def kernel(x, edge_index, W1, b1, W2, b2):
    n, f_in = x.shape
    h = W1.shape[1]
    c_out = W2.shape[1]
    e = edge_index.shape[1]

    n_pad = ((n + 1) + NS * 8 - 1) // (NS * 8) * (NS * 8)
    dump = n
    nch = NS * (K0 + K1)
    assert nch * CHW >= e
    e_pad = (nch + KMAX) * CHW

    pad_col = jnp.concatenate([
        jnp.zeros((1, e_pad - e), jnp.int32),
        jnp.full((1, e_pad - e), dump, jnp.int32),
    ])
    edges = jnp.concatenate([edge_index, pad_col], axis=1).reshape(
        2, nch + KMAX, CHW)

    zeros8 = jnp.zeros((n_pad, 8), jnp.float32)
    ones8 = jnp.ones((CHW, 8), jnp.float32)

    bn = 1000
    grid = (n // bn,)

    h1m = pl.pallas_call(
        _mm_body,
        grid=grid,
        in_specs=[_rows_spec(bn, f_in), _full_spec((f_in, h))],
        out_specs=_rows_spec(bn, h),
        out_shape=jax.ShapeDtypeStruct((n, h), jnp.float32),
    )(x, W1)

    degp = _make_degree_kernel(n_pad)(edges, ones8, zeros8)
    d0 = degp[0, :n]
    d1 = degp[1, :n]

    g1 = pl.pallas_call(
        _tc1_body,
        grid=grid,
        in_specs=[
            _rows_spec(bn, h),
            _rows_spec(bn, 8),
            _rows_spec(bn, 8),
        ],
        out_specs=_rows_spec(bn, h),
        out_shape=jax.ShapeDtypeStruct((n, h), jnp.float32),
    )(h1m, d0, d1)

    zeros_h = jnp.zeros((n_pad, h), jnp.float32)
    sp1 = _make_segsum_kernel(n_pad, h)(g1, edges, zeros_h)

    g2 = pl.pallas_call(
        _tc2_body,
        grid=grid,
        in_specs=[
            _rows_spec(bn, h),
            _rows_spec(bn, h),
            _rows_spec(bn, h),
            _rows_spec(bn, 8),
            _rows_spec(bn, 8),
            _full_spec((1, h)),
            _full_spec((h, c_out)),
        ],
        out_specs=_rows_spec(bn, c_out),
        out_shape=jax.ShapeDtypeStruct((n, c_out), jnp.float32),
    )(sp1[0, :n], sp1[1, :n], g1, d0, d1, b1.reshape(1, h), W2)

    zeros_c = jnp.zeros((n_pad, c_out), jnp.float32)
    sp2 = _make_segsum_kernel(n_pad, c_out)(g2, edges, zeros_c)

    out = pl.pallas_call(
        _tc3_body,
        grid=grid,
        in_specs=[
            _rows_spec(bn, c_out),
            _rows_spec(bn, c_out),
            _rows_spec(bn, c_out),
            _rows_spec(bn, 8),
            _rows_spec(bn, 8),
            _full_spec((1, c_out)),
        ],
        out_specs=_rows_spec(bn, c_out),
        out_shape=jax.ShapeDtypeStruct((n, c_out), jnp.float32),
    )(sp2[0, :n], sp2[1, :n], g2, d0, d1, b2.reshape(1, c_out))

    return out


# speedup vs baseline: 1.3377x; 1.0459x over previous
import jax
import jax.numpy as jnp
from jax import lax
from jax.experimental import pallas as pl
from jax.experimental.pallas import tpu as pltpu
from jax.experimental.pallas import tpu_sc as plsc

NC = 2
NS = 16
NW = NC * NS
CHW = 128

K0 = 84
K1 = 73
KMAX = max(K0, K1)


def _sc_mesh():
    return plsc.VectorSubcoreMesh(core_axis_name="c", subcore_axis_name="s")


def _make_degree_kernel(n_pad):
    def body(edges_hbm, ones_hbm, zeros_hbm, out_hbm, dstv, onesv, acc):
        c = lax.axis_index("c")
        s = lax.axis_index("s")
        k = jnp.where(c == 0, K0, K1)
        start = jnp.where(c == 0, s * K0, NS * K0 + s * K1)
        r = n_pad // NS
        pltpu.sync_copy(zeros_hbm.at[pl.ds(s * r, r)], acc.at[pl.ds(s * r, r)])
        pltpu.sync_copy(edges_hbm.at[1, pl.ds(start, KMAX)], dstv)
        pltpu.sync_copy(ones_hbm, onesv)
        plsc.subcore_barrier()

        def chunk(j, carry):
            @pl.when(j < k)
            def _():
                pltpu.sync_copy(onesv, acc.at[dstv.at[j]], add=True)
            return carry

        lax.fori_loop(0, KMAX, chunk, 0)
        plsc.subcore_barrier()
        pltpu.sync_copy(acc.at[pl.ds(s * r, r)], out_hbm.at[c, pl.ds(s * r, r)])

    return pl.kernel(
        body,
        out_type=jax.ShapeDtypeStruct((NC, n_pad, 8), jnp.float32),
        mesh=_sc_mesh(),
        compiler_params=pltpu.CompilerParams(use_tc_tiling_on_sc=False),
        scratch_types=[
            pltpu.VMEM((KMAX, CHW), jnp.int32),
            pltpu.VMEM((CHW, 8), jnp.float32),
            pltpu.VMEM_SHARED((n_pad, 8), jnp.float32),
        ],
    )


def _make_segsum_kernel(n_pad, f):
    def body(g_hbm, edges_hbm, zeros_hbm, out_hbm, srcv, dstv, msg, acc):
        c = lax.axis_index("c")
        s = lax.axis_index("s")
        k = jnp.where(c == 0, K0, K1)
        start = jnp.where(c == 0, s * K0, NS * K0 + s * K1)
        r = n_pad // NS
        pltpu.sync_copy(zeros_hbm.at[pl.ds(s * r, r)], acc.at[pl.ds(s * r, r)])
        pltpu.sync_copy(edges_hbm.at[0, pl.ds(start, KMAX)], srcv)
        pltpu.sync_copy(edges_hbm.at[1, pl.ds(start, KMAX)], dstv)
        plsc.subcore_barrier()

        def chunk(j, carry):
            @pl.when(j < k)
            def _():
                pltpu.sync_copy(g_hbm.at[srcv.at[j]], msg)
                pltpu.sync_copy(msg, acc.at[dstv.at[j]], add=True)
            return carry

        lax.fori_loop(0, KMAX, chunk, 0)
        plsc.subcore_barrier()
        pltpu.sync_copy(acc.at[pl.ds(s * r, r)], out_hbm.at[c, pl.ds(s * r, r)])

    return pl.kernel(
        body,
        out_type=jax.ShapeDtypeStruct((NC, n_pad, f), jnp.float32),
        mesh=_sc_mesh(),
        compiler_params=pltpu.CompilerParams(use_tc_tiling_on_sc=False),
        scratch_types=[
            pltpu.VMEM((KMAX, CHW), jnp.int32),
            pltpu.VMEM((KMAX, CHW), jnp.int32),
            pltpu.VMEM((CHW, f), jnp.float32),
            pltpu.VMEM_SHARED((n_pad, f), jnp.float32),
        ],
    )


def _dinv(d0_ref, d1_ref):
    deg = d0_ref[:, 0:1] + d1_ref[:, 0:1] + 1.0
    return lax.rsqrt(jnp.maximum(deg, 1.0))


def _mm_body(x_ref, w_ref, h_ref):
    h_ref[...] = jnp.dot(x_ref[...], w_ref[...],
                         preferred_element_type=jnp.float32)


def _tc1_body(h_ref, d0_ref, d1_ref, g_ref):
    g_ref[...] = h_ref[...] * _dinv(d0_ref, d1_ref)


def _tc2_body(p0_ref, p1_ref, g1_ref, d0_ref, d1_ref, b1_ref, w2_ref, g2_ref):
    dinv = _dinv(d0_ref, d1_ref)
    h1 = jnp.maximum(
        dinv * (p0_ref[...] + p1_ref[...] + g1_ref[...]) + b1_ref[...], 0.0)
    g2_ref[...] = jnp.dot(h1, w2_ref[...],
                          preferred_element_type=jnp.float32) * dinv


def _tc3_body(q0_ref, q1_ref, g2_ref, d0_ref, d1_ref, b2_ref, o_ref):
    t = (_dinv(d0_ref, d1_ref) * (q0_ref[...] + q1_ref[...] + g2_ref[...])
         + b2_ref[...])
    m = jnp.max(t, axis=1, keepdims=True)
    lse = m + jnp.log(jnp.sum(jnp.exp(t - m), axis=1, keepdims=True))
    o_ref[...] = t - lse


def _rows_spec(bn, cols):
    return pl.BlockSpec((bn, cols), lambda i: (i, 0))


def _full_spec(shape):
    return pl.BlockSpec(shape, lambda i: tuple(0 for _ in shape))


@jax.jit
def kernel(x, edge_index, W1, b1, W2, b2):
    n, f_in = x.shape
    h = W1.shape[1]
    c_out = W2.shape[1]
    e = edge_index.shape[1]

    n_pad = ((n + 1) + NS * 8 - 1) // (NS * 8) * (NS * 8)
    dump = n
    nch = NS * (K0 + K1)
    assert nch * CHW >= e
    e_pad = (nch + KMAX) * CHW

    pad_col = jnp.concatenate([
        jnp.zeros((1, e_pad - e), jnp.int32),
        jnp.full((1, e_pad - e), dump, jnp.int32),
    ])
    edges = jnp.concatenate([edge_index, pad_col], axis=1).reshape(
        2, nch + KMAX, CHW)

    zeros8 = jnp.zeros((n_pad, 8), jnp.float32)
    ones8 = jnp.ones((CHW, 8), jnp.float32)

    bn = 1000
    grid = (n // bn,)

    h1m = pl.pallas_call(
        _mm_body,
        grid=grid,
        in_specs=[_rows_spec(bn, f_in), _full_spec((f_in, h))],
        out_specs=_rows_spec(bn, h),
        out_shape=jax.ShapeDtypeStruct((n, h), jnp.float32),
    )(x, W1)

    degp = _make_degree_kernel(n_pad)(edges, ones8, zeros8)
    d0 = degp[0, :n]
    d1 = degp[1, :n]

    g1 = pl.pallas_call(
        _tc1_body,
        grid=grid,
        in_specs=[
            _rows_spec(bn, h),
            _rows_spec(bn, 8),
            _rows_spec(bn, 8),
        ],
        out_specs=_rows_spec(bn, h),
        out_shape=jax.ShapeDtypeStruct((n, h), jnp.float32),
    )(h1m, d0, d1)

    zeros_h = jnp.zeros((n_pad, h), jnp.float32)
    sp1 = _make_segsum_kernel(n_pad, h)(g1, edges, zeros_h)

    g2 = pl.pallas_call(
        _tc2_body,
        grid=grid,
        in_specs=[
            _rows_spec(bn, h),
            _rows_spec(bn, h),
            _rows_spec(bn, h),
            _rows_spec(bn, 8),
            _rows_spec(bn, 8),
            _full_spec((1, h)),
            _full_spec((h, c_out)),
        ],
        out_specs=_rows_spec(bn, c_out),
        out_shape=jax.ShapeDtypeStruct((n, c_out), jnp.float32),
    )(sp1[0, :n], sp1[1, :n], g1, d0, d1, b1.reshape(1, h), W2)

    zeros_c = jnp.zeros((n_pad, c_out), jnp.float32)
    sp2 = _make_segsum_kernel(n_pad, c_out)(g2, edges, zeros_c)

    out = pl.pallas_call(
        _tc3_body,
        grid=grid,
        in_specs=[
            _rows_spec(bn, c_out),
            _rows_spec(bn, c_out),
            _rows_spec(bn, c_out),
            _rows_spec(bn, 8),
            _rows_spec(bn, 8),
            _full_spec((1, c_out)),
        ],
        out_specs=_rows_spec(bn, c_out),
        out_shape=jax.ShapeDtypeStruct((n, c_out), jnp.float32),
    )(sp2[0, :n], sp2[1, :n], g2, d0, d1, b2.reshape(1, c_out))

    return out
